# baseline (speedup 1.0000x reference)
import functools

import jax
import jax.numpy as jnp
from jax import lax
from jax.experimental import pallas as pl
from jax.experimental.pallas import tpu as pltpu
from jax.experimental.pallas import tpu_sc as plsc

BATCH = 16384
NFIELDS = 26
TOTAL_ROWS = 26_000_000
PAD_ROWS = 384
NC = 2
NS = 16
NW = NC * NS
BPW = BATCH // NW
EPW = NFIELDS * BPW
JCH = BPW // 16


def _make_kernel():
    mesh = plsc.VectorSubcoreMesh(core_axis_name="c", subcore_axis_name="s")

    @functools.partial(
        pl.kernel,
        mesh=mesh,
        out_type=jax.ShapeDtypeStruct((BATCH,), jnp.float32),
        scratch_types=[
            pltpu.VMEM((EPW,), jnp.int32),
            pltpu.VMEM((NFIELDS * 16,), jnp.int32),
            pltpu.VMEM((EPW,), jnp.float32),
            pltpu.VMEM((16,), jnp.float32),
            pltpu.VMEM((BPW,), jnp.float32),
            pltpu.SemaphoreType.DMA,
        ],
    )
    def k(xw_hbm, offs_hbm, table_hbm, bias_hbm, out_hbm,
          idx_v, off_v, val_v, bias_v, out_v, sem):
        wid = lax.axis_index("s") * NC + lax.axis_index("c")
        with jax.named_scope("dma_in"):
            pltpu.sync_copy(xw_hbm.at[wid], idx_v)
            pltpu.sync_copy(offs_hbm, off_v)
            pltpu.sync_copy(bias_hbm, bias_v)

        offv = [off_v[pl.ds(f * 16, 16)] for f in range(NFIELDS)]

        with jax.named_scope("add_loop"):
            def add_body(j, carry):
                c = j * 16
                for f in range(NFIELDS):
                    s = f * BPW + c
                    idx_v[pl.ds(s, 16)] = idx_v[pl.ds(s, 16)] + offv[f]
                return carry
            lax.fori_loop(0, JCH, add_body, 0)

        with jax.named_scope("gather"):
            pltpu.async_copy(table_hbm.at[idx_v], val_v, sem).wait()

        bvec = bias_v[...]

        with jax.named_scope("reduce"):
            def red_body(j, carry):
                c = j * 16
                acc = bvec
                for f in range(NFIELDS):
                    acc = acc + val_v[pl.ds(f * BPW + c, 16)]
                out_v[pl.ds(c, 16)] = acc
                return carry
            lax.fori_loop(0, JCH, red_body, 0)

        with jax.named_scope("dma_out"):
            pltpu.sync_copy(out_v, out_hbm.at[pl.ds(wid * BPW, BPW)])

    return k


_gather_sum = _make_kernel()


def kernel(x, offsets, table, bias):
    xw = (x.T.reshape(NFIELDS, NW, BPW)
          .transpose(1, 0, 2)
          .reshape(NW, EPW))
    offs = jnp.repeat(offsets, 16)
    table_flat = jnp.pad(table, ((0, PAD_ROWS), (0, 0))).reshape(
        TOTAL_ROWS + PAD_ROWS)
    bias16 = jnp.broadcast_to(bias.astype(jnp.float32), (16,))
    out = _gather_sum(xw, offs, table_flat, bias16)
    return out.reshape(BATCH, 1)


# speedup vs baseline: 1.0196x; 1.0196x over previous
import functools

import jax
import jax.numpy as jnp
from jax import lax
from jax.experimental import pallas as pl
from jax.experimental.pallas import tpu as pltpu
from jax.experimental.pallas import tpu_sc as plsc

BATCH = 16384
NFIELDS = 26
TOTAL_ROWS = 26_000_000
PAD_ROWS = 384
NC = 2
NS = 16
NW = NC * NS
BPW = BATCH // NW
EPW = NFIELDS * BPW
JCH = BPW // 16


def _make_kernel():
    mesh = plsc.VectorSubcoreMesh(core_axis_name="c", subcore_axis_name="s")

    @functools.partial(
        pl.kernel,
        mesh=mesh,
        out_type=jax.ShapeDtypeStruct((BATCH,), jnp.float32),
        scratch_types=[
            pltpu.VMEM((NFIELDS, BPW), jnp.int32),
            pltpu.VMEM((EPW,), jnp.int32),
            pltpu.VMEM((NFIELDS * 16,), jnp.int32),
            pltpu.VMEM((EPW,), jnp.float32),
            pltpu.VMEM((16,), jnp.float32),
            pltpu.VMEM((BPW,), jnp.float32),
            pltpu.SemaphoreType.DMA,
        ],
    )
    def k(xt_hbm, offs_hbm, table_hbm, bias_hbm, out_hbm,
          x_v, idx_v, off_v, val_v, bias_v, out_v, sem):
        wid = lax.axis_index("s") * NC + lax.axis_index("c")
        with jax.named_scope("dma_in"):
            cp_x = pltpu.async_copy(
                xt_hbm.at[:, pl.ds(wid * BPW, BPW)], x_v, sem)
            cp_o = pltpu.async_copy(offs_hbm, off_v, sem)
            cp_b = pltpu.async_copy(bias_hbm, bias_v, sem)
            cp_x.wait()
            cp_o.wait()
            cp_b.wait()

        offv = [off_v[pl.ds(f * 16, 16)] for f in range(NFIELDS)]

        with jax.named_scope("add_loop"):
            def add_body(j, carry):
                c = j * 16
                for f in range(NFIELDS):
                    idx_v[pl.ds(f * BPW + c, 16)] = (
                        x_v[f, pl.ds(c, 16)] + offv[f])
                return carry
            lax.fori_loop(0, JCH, add_body, 0)

        with jax.named_scope("gather"):
            pltpu.async_copy(table_hbm.at[idx_v], val_v, sem).wait()

        bvec = bias_v[...]

        with jax.named_scope("reduce"):
            def red_body(j, carry):
                c = j * 16
                acc = bvec
                for f in range(NFIELDS):
                    acc = acc + val_v[pl.ds(f * BPW + c, 16)]
                out_v[pl.ds(c, 16)] = acc
                return carry
            lax.fori_loop(0, JCH, red_body, 0)

        with jax.named_scope("dma_out"):
            pltpu.sync_copy(out_v, out_hbm.at[pl.ds(wid * BPW, BPW)])

    return k


_gather_sum = _make_kernel()


def kernel(x, offsets, table, bias):
    xt = x.T
    offs = jnp.repeat(offsets, 16)
    table_flat = jnp.concatenate(
        [table, jnp.zeros((PAD_ROWS, 1), jnp.float32)]).reshape(
        TOTAL_ROWS + PAD_ROWS)
    bias16 = jnp.broadcast_to(bias.astype(jnp.float32), (16,))
    out = _gather_sum(xt, offs, table_flat, bias16)
    return out.reshape(BATCH, 1)
